# baseline (speedup 1.0000x reference)
import jax
import jax.numpy as jnp
from jax import lax
from jax.experimental import pallas as pl
from jax.experimental.pallas import tpu as pltpu
from jax.experimental.pallas import tpu_sc as plsc

N = 10000
E = 320000
F = 128
FE = 16
G = 100
L = 3
EPS = 1e-5

NC, NS = 2, 16
NW = NC * NS
CH = 80
NOPS = 125
RB = 80
RPT = 640

HP = lax.Precision.HIGHEST


def _dotm(a, b, dims=None):
    if dims is None:
        dims = (((a.ndim - 1,), (0,)), ((), ()))
    return lax.dot_general(a.astype(jnp.bfloat16), b.astype(jnp.bfloat16),
                           dims, preferred_element_type=jnp.float32)


def _bn(y, g, b):
    mu = jnp.mean(y, axis=0, keepdims=True)
    var = jnp.mean((y - mu) * (y - mu), axis=0, keepdims=True)
    return (y - mu) * (1.0 / jnp.sqrt(var + EPS)) * g + b


def _stats_body(ea_ref, sum_ref, mom_ref):
    i = pl.program_id(0)

    @pl.when(i == 0)
    def _():
        sum_ref[...] = jnp.zeros_like(sum_ref)
        mom_ref[...] = jnp.zeros_like(mom_ref)

    ea = ea_ref[...].astype(jnp.bfloat16).astype(jnp.float32)
    sum_ref[...] += jnp.sum(ea, axis=0, keepdims=True)
    mom_ref[...] += _dotm(ea, ea, (((0,), (0,)), ((), ())))


def _edge_stats(ea):
    nb = 64
    bs = E // nb
    return pl.pallas_call(
        _stats_body,
        grid=(nb,),
        in_specs=[pl.BlockSpec((bs, FE), lambda i: (i, 0))],
        out_specs=[pl.BlockSpec((1, FE), lambda i: (0, 0)),
                   pl.BlockSpec((FE, FE), lambda i: (0, 0))],
        out_shape=[jax.ShapeDtypeStruct((1, FE), jnp.float32),
                   jax.ShapeDtypeStruct((FE, FE), jnp.float32)],
    )(ea)


def _h0_body(x_ref, w_ref, g_ref, b_ref, o_ref):
    y = _dotm(x_ref[...], w_ref[...])
    o_ref[...] = jnp.maximum(_bn(y, g_ref[...], b_ref[...]), 0.0)


def _h0(x, w, g, b):
    return pl.pallas_call(
        _h0_body,
        out_shape=jax.ShapeDtypeStruct((N, F), jnp.float32),
    )(x, w, g.reshape(1, F), b.reshape(1, F))


def _enc_body(ea_ref, sum_ref, mom_ref, we_ref, ge_ref, be_ref, e_ref):
    ea = ea_ref[...]
    mean_a = sum_ref[...] * (1.0 / E)
    cov = mom_ref[...] * (1.0 / E) - lax.dot_general(
        mean_a, mean_a, (((0,), (0,)), ((), ())),
        preferred_element_type=jnp.float32, precision=HP)
    we = we_ref[...]
    weq = we.astype(jnp.bfloat16).astype(jnp.float32)
    mu = jnp.dot(mean_a, weq, preferred_element_type=jnp.float32,
                 precision=HP)
    var = jnp.sum(weq * jnp.dot(cov, weq, preferred_element_type=jnp.float32,
                                precision=HP),
                  axis=0, keepdims=True)
    s = ge_ref[...] * (1.0 / jnp.sqrt(var + EPS))
    t = be_ref[...] - mu * s
    y = _dotm(ea, we)
    e_ref[...] = jnp.maximum(y * s + t, 0.0)


def _edge_enc(ea, sum_a, mom, we, ge, be):
    nb = 64
    bs = E // nb
    return pl.pallas_call(
        _enc_body,
        grid=(nb,),
        in_specs=[
            pl.BlockSpec((bs, FE), lambda i: (i, 0)),
            pl.BlockSpec((1, FE), lambda i: (0, 0)),
            pl.BlockSpec((FE, FE), lambda i: (0, 0)),
            pl.BlockSpec((FE, F), lambda i: (0, 0)),
            pl.BlockSpec((1, F), lambda i: (0, 0)),
            pl.BlockSpec((1, F), lambda i: (0, 0)),
        ],
        out_specs=pl.BlockSpec((bs, F), lambda i: (i, 0)),
        out_shape=jax.ShapeDtypeStruct((E, F), jnp.float32),
    )(ea, sum_a, mom, we, ge, be)


def _sc_edge_body(h_hbm, src_hbm, dst_hbm, enc_hbm, out_hbm,
                  ssrc, idx_d, hbuf, ebuf, acc,
                  isem, jsem, gsem, esem, shsem):
    c = lax.axis_index("c")
    s = lax.axis_index("s")
    w = c * NS + s
    ebase = w * NOPS * CH

    nkr = jnp.where(s < NS - 1, RPT // RB, (N - (NS - 1) * RPT) // RB)
    zb = hbuf[0]

    def zrow(r, _):
        for k in range(F // 16):
            zb[r, pl.ds(k * 16, 16)] = jnp.zeros((16,), jnp.float32)
        return 0
    lax.fori_loop(0, RB, zrow, 0)

    def zcp(k, _):
        pltpu.sync_copy(zb, acc.at[pl.ds(s * RPT + k * RB, RB), :])
        return 0
    lax.fori_loop(0, nkr, zcp, 0)
    plsc.subcore_barrier()

    def issue_src(j, b):
        pltpu.async_copy(src_hbm.at[pl.ds(ebase + j * CH, CH)],
                         ssrc[b], isem[b])

    def issue_dst(j, b4):
        pltpu.async_copy(dst_hbm.at[pl.ds(ebase + j * CH, CH)],
                         idx_d[b4], jsem[b4])

    def issue_ge(j, b):
        pltpu.async_copy(h_hbm.at[ssrc[b]], hbuf[b], gsem[b])
        pltpu.async_copy(enc_hbm.at[pl.ds(ebase + j * CH, CH), :],
                         ebuf[b], esem[b])

    def wait_ge(b):
        pltpu.make_async_copy(h_hbm.at[ssrc[b]], hbuf[b], gsem[b]).wait()
        pltpu.make_async_copy(enc_hbm.at[pl.ds(0, CH), :],
                              ebuf[b], esem[b]).wait()

    def wait_idx(sem, ref):
        pltpu.make_async_copy(src_hbm.at[pl.ds(0, CH)], ref, sem).wait()

    def drain_sc(b):
        pltpu.make_async_copy(hbuf[b], acc.at[idx_d[0]], shsem[b]).wait()

    def addeb(b):
        @plsc.parallel_loop(0, CH, 1, unroll=2)
        def _(r):
            for k in range(F // 16):
                sl = pl.ds(k * 16, 16)
                hbuf[b][r, sl] = hbuf[b][r, sl] + ebuf[b][r, sl]

    issue_src(0, 0)
    issue_src(1, 1)
    issue_dst(0, 0)
    issue_dst(1, 1)
    wait_idx(isem[0], ssrc[0])
    issue_ge(0, 0)

    def group(g, _):
        for i in range(4):
            b = i % 2
            bo = 1 - b
            j = g * 4 + i
            wait_idx(isem[bo], ssrc[bo])

            @pl.when(j >= 1)
            def _():
                drain_sc(bo)
            issue_ge(j + 1, bo)
            wait_ge(b)

            @pl.when(j + 2 < NOPS)
            def _():
                issue_src(j + 2, b)
                issue_dst(j + 2, (i + 2) % 4)
            addeb(b)
            wait_idx(jsem[i], idx_d[i])
            pltpu.async_copy(hbuf[b], acc.at[idx_d[i]], shsem[b], add=True)
        return 0

    lax.fori_loop(0, (NOPS - 1) // 4, group, 0)

    wait_ge(0)
    addeb(0)
    wait_idx(jsem[0], idx_d[0])
    pltpu.async_copy(hbuf[0], acc.at[idx_d[0]], shsem[0], add=True)
    drain_sc(1)
    drain_sc(0)

    plsc.subcore_barrier()

    def ocp(k, _):
        r0 = s * RPT + k * RB
        pltpu.sync_copy(acc.at[pl.ds(r0, RB), :], zb)
        pltpu.sync_copy(zb, out_hbm.at[c, pl.ds(r0, RB), :])
        return 0
    lax.fori_loop(0, nkr, ocp, 0)


def _sc_edge(h, src, dst, enc):
    return pl.kernel(
        _sc_edge_body,
        out_type=jax.ShapeDtypeStruct((NC, N, F), jnp.float32),
        mesh=plsc.VectorSubcoreMesh(core_axis_name="c", subcore_axis_name="s"),
        scratch_types=[
            [pltpu.VMEM((CH,), jnp.int32) for _ in range(2)],
            [pltpu.VMEM((CH,), jnp.int32) for _ in range(4)],
            [pltpu.VMEM((CH, F), jnp.float32) for _ in range(2)],
            [pltpu.VMEM((CH, F), jnp.float32) for _ in range(2)],
            pltpu.VMEM_SHARED((N, F), jnp.float32),
            [pltpu.SemaphoreType.DMA for _ in range(2)],
            [pltpu.SemaphoreType.DMA for _ in range(4)],
            [pltpu.SemaphoreType.DMA for _ in range(2)],
            [pltpu.SemaphoreType.DMA for _ in range(2)],
            [pltpu.SemaphoreType.DMA for _ in range(2)],
        ],
    )(h, src, dst, enc)


def _node_body(h_ref, p_ref, w1_ref, g1_ref, b1_ref,
               w2_ref, g2_ref, b2_ref, gn_ref, bn_ref, o_ref):
    z = h_ref[...] + p_ref[0] + p_ref[1]
    z = jnp.maximum(_bn(_dotm(z, w1_ref[...]),
                        g1_ref[...], b1_ref[...]), 0.0)
    z = jnp.maximum(_bn(_dotm(z, w2_ref[...]),
                        g2_ref[...], b2_ref[...]), 0.0)
    o_ref[...] = jnp.maximum(_bn(z, gn_ref[...], bn_ref[...]), 0.0)


def _node(h, parts, lp):
    r = lambda v: v.reshape(1, F)
    return pl.pallas_call(
        _node_body,
        out_shape=jax.ShapeDtypeStruct((N, F), jnp.float32),
    )(h, parts, lp["Wc1"], r(lp["gc1"]), r(lp["bc1"]),
      lp["Wc2"], r(lp["gc2"]), r(lp["bc2"]), r(lp["gn"]), r(lp["bn"]))


def _pool_body(h1_ref, h2_ref, h3_ref, batch_ref, emb_ref,
               w1_ref, b1_ref, w2_ref, b2_ref, o_ref):
    gids = lax.broadcasted_iota(jnp.int32, (1, G), 1)
    onehot = (batch_ref[...] == gids).astype(jnp.float32)
    counts = jnp.sum(onehot, axis=0, keepdims=True)
    segs = [lax.dot_general(onehot, h, (((0,), (0,)), ((), ())),
                            preferred_element_type=jnp.float32, precision=HP)
            for h in (h1_ref[...], h2_ref[...], h3_ref[...])]
    mean = jnp.concatenate(segs, axis=1) / jnp.maximum(
        counts, 1.0).reshape(G, 1)
    cnt = jnp.minimum(counts.reshape(G, 1), 199.0).astype(jnp.int32)
    sel = (cnt == lax.broadcasted_iota(jnp.int32, (1, 200), 1)
           ).astype(jnp.float32)
    pooled = mean + jnp.dot(sel, emb_ref[...],
                            preferred_element_type=jnp.float32, precision=HP)
    hh = jnp.maximum(_dotm(pooled, w1_ref[...]) + b1_ref[...], 0.0)
    o_ref[...] = _dotm(hh, w2_ref[...]) + b2_ref[...]


def _pool(h1, h2, h3, batch, emb, w1, b1, w2, b2):
    return pl.pallas_call(
        _pool_body,
        out_shape=jax.ShapeDtypeStruct((G, 1), jnp.float32),
    )(h1, h2, h3, batch.reshape(N, 1), emb,
      w1, b1.reshape(1, 3 * F), w2, b2.reshape(1, 1))


def kernel(x, edge_index, edge_attr, batch, params):
    sum_a, mom = _edge_stats(edge_attr)
    h = _h0(x, params["W_in"], params["g_in"], params["b_in"])
    lps = params["layers"]

    def enc_l(l):
        lp = lps[l]
        return _edge_enc(edge_attr, sum_a, mom, lp["We"],
                         lp["ge"].reshape(1, F), lp["be"].reshape(1, F))

    src, dst = edge_index[0], edge_index[1]
    skips = []
    enc = enc_l(0)
    for l in range(L):
        parts = _sc_edge(h, src, dst, enc)
        if l + 1 < L:
            enc = enc_l(l + 1)
        h = _node(h, parts, lps[l])
        skips.append(h)

    return _pool(skips[0], skips[1], skips[2], batch, params["size_emb"],
                 params["Wo1"], params["bo1"], params["Wo2"], params["bo2"])
